# baseline (speedup 1.0000x reference)
import functools

import jax
import jax.numpy as jnp
from jax import lax
from jax.experimental import pallas as pl
from jax.experimental.pallas import tpu as pltpu

_N = 10000
_E = 320000
_H = 128
_MID = 256
_F1 = 54
_G = 64


def _edge_body(f1_ref, gate_ref, wa_ref, wb_ref, m1_ref, m2_ref):
    q = jnp.dot(f1_ref[...], wa_ref[...], preferred_element_type=jnp.float32)
    f = jnp.dot(q, wb_ref[...], preferred_element_type=jnp.float32)
    g = gate_ref[...]
    m1 = f * g
    m1_ref[...] = m1
    m2_ref[...] = m1 * g


def _edge_stage(feature1, gate_edge, W_f1a, W_f1b):
    BE = 2000
    return pl.pallas_call(
        _edge_body,
        grid=(_E // BE,),
        in_specs=[
            pl.BlockSpec((BE, _F1), lambda i: (i, 0)),
            pl.BlockSpec((BE, _H), lambda i: (i, 0)),
            pl.BlockSpec((_F1, _MID), lambda i: (0, 0)),
            pl.BlockSpec((_MID, _H), lambda i: (0, 0)),
        ],
        out_specs=[
            pl.BlockSpec((BE, _H), lambda i: (i, 0)),
            pl.BlockSpec((BE, _H), lambda i: (i, 0)),
        ],
        out_shape=[jax.ShapeDtypeStruct((_E, _H), jnp.float32)] * 2,
    )(feature1, gate_edge, W_f1a, W_f1b)


def _leaky(v):
    return jnp.where(v > 0, v, 0.1 * v)


def _node_body(x_ref, a1_ref, a2_ref, batch_ref,
               wr1_ref, br1_ref, wo1_ref, wr2_ref, br2_ref, wo2_ref,
               wl1_ref, bl1_ref, wl2_ref, bl2_ref, wc_ref, bc_ref,
               gw_ref, gb_ref, gms_ref, out_ref):
    x = x_ref[...]
    dot = functools.partial(jnp.dot, preferred_element_type=jnp.float32)
    h1 = dot(a1_ref[...], wr1_ref[...]) + br1_ref[...] + dot(x, wo1_ref[...])
    h1 = _leaky(dot(h1, wl1_ref[...]) + bl1_ref[...])
    h2 = dot(a2_ref[...], wr2_ref[...]) + br2_ref[...] + dot(x, wo2_ref[...])
    h2 = _leaky(dot(h2, wl2_ref[...]) + bl2_ref[...])
    wc = wc_ref[...]
    h = dot(h1, wc[:_H, :]) + dot(h2, wc[_H:, :]) + bc_ref[...]

    batch = batch_ref[...]
    seg = lax.broadcasted_iota(jnp.int32, (_G, _N), 0)
    onehot = (seg == batch).astype(jnp.float32)
    cnt = jnp.maximum(jnp.sum(onehot, axis=1, keepdims=True), 1.0)
    mean = dot(onehot, h) / cnt
    mean_n = lax.dot_general(onehot, mean, (((0,), (0,)), ((), ())),
                             preferred_element_type=jnp.float32)
    out = h - mean_n * gms_ref[...]
    var = dot(onehot, out * out) / cnt
    var_n = lax.dot_general(onehot, var, (((0,), (0,)), ((), ())),
                            preferred_element_type=jnp.float32)
    out_ref[...] = gw_ref[...] * out * lax.rsqrt(var_n + 1e-5) + gb_ref[...]


def _node_stage(x, agg1, agg2, batch,
                W_rel1, b_rel1, W_root1, W_rel2, b_rel2, W_root2,
                W_lin1, b_lin1, W_lin2, b_lin2, W_cat, b_cat,
                gn_weight, gn_bias, gn_mean_scale):
    row = lambda v: v.reshape(1, -1)

    def full(a):
        nd = a.ndim
        return pl.BlockSpec(a.shape, lambda: (0,) * nd)

    args = (x, agg1, agg2, batch.reshape(1, _N),
            W_rel1, row(b_rel1), W_root1, W_rel2, row(b_rel2), W_root2,
            W_lin1, row(b_lin1), W_lin2, row(b_lin2), W_cat, row(b_cat),
            row(gn_weight), row(gn_bias), row(gn_mean_scale))
    return pl.pallas_call(
        _node_body,
        in_specs=[full(a) for a in args],
        out_specs=pl.BlockSpec((_N, _H), lambda: (0, 0)),
        out_shape=jax.ShapeDtypeStruct((_N, _H), jnp.float32),
    )(*args)


def kernel(x, gate_edge, feature1, feature2, edge_index, batch,
           W_f1a, W_f1b, W_f2a, W_f2b,
           W_rel1, b_rel1, W_root1, W_rel2, b_rel2, W_root2,
           W_lin1, b_lin1, W_lin2, b_lin2, W_cat, b_cat,
           gn_weight, gn_bias, gn_mean_scale):
    src, dst = edge_index[0], edge_index[1]
    m1, m2 = _edge_stage(feature1, gate_edge, W_f1a, W_f1b)
    xg = x[src]
    agg1 = jax.ops.segment_sum(xg * m1, dst, num_segments=_N)
    agg2 = jax.ops.segment_sum(xg * m2, dst, num_segments=_N)
    return _node_stage(x, agg1, agg2, batch,
                       W_rel1, b_rel1, W_root1, W_rel2, b_rel2, W_root2,
                       W_lin1, b_lin1, W_lin2, b_lin2, W_cat, b_cat,
                       gn_weight, gn_bias, gn_mean_scale)


# speedup vs baseline: 2.4366x; 2.4366x over previous
import functools

import jax
import jax.numpy as jnp
from jax import lax
from jax.experimental import pallas as pl
from jax.experimental.pallas import tpu as pltpu
from jax.experimental.pallas import tpu_sc as plsc

_N = 10000
_E = 320000
_H = 128
_MID = 256
_F1 = 54
_G = 64

_NC = 2
_NS = 16
_CB = 128
_NCHUNK = _E // _CB
_KMAX = -(-_NCHUNK // _NS)
_RG = 80
_NRG = _N // _RG
_KRG = -(-_NRG // _NS)


def _sc_scatter_body(src_hbm, dst_hbm, m_hbm, x_hbm, out_hbm,
                     srcv, dstv, xrows, mrows, agg, sem):
    c = lax.axis_index("c")
    s = lax.axis_index("s")

    zero = jnp.zeros((16,), jnp.float32)

    def zero_row(r, _):
        for j in range(8):
            xrows[r, pl.ds(j * 16, 16)] = zero
        return _

    lax.fori_loop(0, _RG, zero_row, None)

    def zero_grp(k, _):
        g = k * _NS + s

        @pl.when(g < _NRG)
        def _():
            pltpu.sync_copy(xrows.at[pl.ds(0, _RG)],
                            agg.at[pl.ds(g * _RG, _RG)])

        return _

    lax.fori_loop(0, _KRG, zero_grp, None)
    plsc.subcore_barrier()

    def chunk_body(k, _):
        g = k * _NS + s

        @pl.when(g < _NCHUNK)
        def _():
            base = g * _CB
            pltpu.sync_copy(src_hbm.at[pl.ds(base, _CB)], srcv)
            pltpu.sync_copy(dst_hbm.at[pl.ds(base, _CB)], dstv)
            pltpu.sync_copy(m_hbm.at[c, pl.ds(base, _CB)], mrows)
            pltpu.async_copy(x_hbm.at[srcv], xrows, sem).wait()

            def mul_row(r, carry):
                for j in range(8):
                    sl = pl.ds(j * 16, 16)
                    mrows[r, sl] = mrows[r, sl] * xrows[r, sl]
                return carry

            lax.fori_loop(0, _CB, mul_row, None)
            pltpu.sync_copy(mrows, agg.at[dstv], add=True)

        return _

    lax.fori_loop(0, _KMAX, chunk_body, None)
    plsc.subcore_barrier()

    def out_grp(k, _):
        g = k * _NS + s

        @pl.when(g < _NRG)
        def _():
            pltpu.sync_copy(agg.at[pl.ds(g * _RG, _RG)],
                            out_hbm.at[c, pl.ds(g * _RG, _RG)])

        return _

    lax.fori_loop(0, _KRG, out_grp, None)


def _scatter_stage(src, dst, m, x):
    f = pl.kernel(
        _sc_scatter_body,
        out_type=jax.ShapeDtypeStruct((_NC, _N, _H), jnp.float32),
        mesh=plsc.VectorSubcoreMesh(core_axis_name="c", subcore_axis_name="s",
                                    num_cores=_NC, num_subcores=_NS),
        scratch_types=[
            pltpu.VMEM((_CB,), jnp.int32),
            pltpu.VMEM((_CB,), jnp.int32),
            pltpu.VMEM((_CB, _H), jnp.float32),
            pltpu.VMEM((_CB, _H), jnp.float32),
            pltpu.VMEM_SHARED((_N, _H), jnp.float32),
            pltpu.SemaphoreType.DMA,
        ],
    )
    return f(src, dst, m, x)


def _edge_body(f1_ref, gate_ref, wa_ref, wb_ref, m_ref):
    q = jnp.dot(f1_ref[...], wa_ref[...], preferred_element_type=jnp.float32)
    f = jnp.dot(q, wb_ref[...], preferred_element_type=jnp.float32)
    g = gate_ref[...]
    m1 = f * g
    m_ref[0] = m1
    m_ref[1] = m1 * g


def _edge_stage(feature1, gate_edge, W_f1a, W_f1b):
    BE = 2000
    return pl.pallas_call(
        _edge_body,
        grid=(_E // BE,),
        in_specs=[
            pl.BlockSpec((BE, _F1), lambda i: (i, 0)),
            pl.BlockSpec((BE, _H), lambda i: (i, 0)),
            pl.BlockSpec((_F1, _MID), lambda i: (0, 0)),
            pl.BlockSpec((_MID, _H), lambda i: (0, 0)),
        ],
        out_specs=pl.BlockSpec((2, BE, _H), lambda i: (0, i, 0)),
        out_shape=jax.ShapeDtypeStruct((2, _E, _H), jnp.float32),
    )(feature1, gate_edge, W_f1a, W_f1b)


def _leaky(v):
    return jnp.where(v > 0, v, 0.1 * v)


def _node_body(x_ref, a1_ref, a2_ref, batch_ref,
               wr1_ref, br1_ref, wo1_ref, wr2_ref, br2_ref, wo2_ref,
               wl1_ref, bl1_ref, wl2_ref, bl2_ref, wc_ref, bc_ref,
               gw_ref, gb_ref, gms_ref, out_ref):
    x = x_ref[...]
    dot = functools.partial(jnp.dot, preferred_element_type=jnp.float32)
    h1 = dot(a1_ref[...], wr1_ref[...]) + br1_ref[...] + dot(x, wo1_ref[...])
    h1 = _leaky(dot(h1, wl1_ref[...]) + bl1_ref[...])
    h2 = dot(a2_ref[...], wr2_ref[...]) + br2_ref[...] + dot(x, wo2_ref[...])
    h2 = _leaky(dot(h2, wl2_ref[...]) + bl2_ref[...])
    wc = wc_ref[...]
    h = dot(h1, wc[:_H, :]) + dot(h2, wc[_H:, :]) + bc_ref[...]

    batch = batch_ref[...]
    seg = lax.broadcasted_iota(jnp.int32, (_G, _N), 0)
    onehot = (seg == batch).astype(jnp.float32)
    cnt = jnp.maximum(jnp.sum(onehot, axis=1, keepdims=True), 1.0)
    mean = dot(onehot, h) / cnt
    mean_n = lax.dot_general(onehot, mean, (((0,), (0,)), ((), ())),
                             preferred_element_type=jnp.float32)
    out = h - mean_n * gms_ref[...]
    var = dot(onehot, out * out) / cnt
    var_n = lax.dot_general(onehot, var, (((0,), (0,)), ((), ())),
                            preferred_element_type=jnp.float32)
    out_ref[...] = gw_ref[...] * out * lax.rsqrt(var_n + 1e-5) + gb_ref[...]


def _node_stage(x, agg1, agg2, batch,
                W_rel1, b_rel1, W_root1, W_rel2, b_rel2, W_root2,
                W_lin1, b_lin1, W_lin2, b_lin2, W_cat, b_cat,
                gn_weight, gn_bias, gn_mean_scale):
    row = lambda v: v.reshape(1, -1)

    def full(a):
        nd = a.ndim
        return pl.BlockSpec(a.shape, lambda: (0,) * nd)

    args = (x, agg1, agg2, batch.reshape(1, _N),
            W_rel1, row(b_rel1), W_root1, W_rel2, row(b_rel2), W_root2,
            W_lin1, row(b_lin1), W_lin2, row(b_lin2), W_cat, row(b_cat),
            row(gn_weight), row(gn_bias), row(gn_mean_scale))
    return pl.pallas_call(
        _node_body,
        in_specs=[full(a) for a in args],
        out_specs=pl.BlockSpec((_N, _H), lambda: (0, 0)),
        out_shape=jax.ShapeDtypeStruct((_N, _H), jnp.float32),
    )(*args)


def kernel(x, gate_edge, feature1, feature2, edge_index, batch,
           W_f1a, W_f1b, W_f2a, W_f2b,
           W_rel1, b_rel1, W_root1, W_rel2, b_rel2, W_root2,
           W_lin1, b_lin1, W_lin2, b_lin2, W_cat, b_cat,
           gn_weight, gn_bias, gn_mean_scale):
    src, dst = edge_index[0], edge_index[1]
    m = _edge_stage(feature1, gate_edge, W_f1a, W_f1b)
    agg = _scatter_stage(src, dst, m, x)
    agg1, agg2 = agg[0], agg[1]
    return _node_stage(x, agg1, agg2, batch,
                       W_rel1, b_rel1, W_root1, W_rel2, b_rel2, W_root2,
                       W_lin1, b_lin1, W_lin2, b_lin2, W_cat, b_cat,
                       gn_weight, gn_bias, gn_mean_scale)


# speedup vs baseline: 2.4534x; 1.0069x over previous
import functools

import jax
import jax.numpy as jnp
from jax import lax
from jax.experimental import pallas as pl
from jax.experimental.pallas import tpu as pltpu
from jax.experimental.pallas import tpu_sc as plsc

_N = 10000
_E = 320000
_H = 128
_MID = 256
_F1 = 54
_G = 64

_NC = 2
_NS = 16
_CB = 128
_NCHUNK = _E // _CB
_KMAX = -(-_NCHUNK // _NS)
_AROWS = _N
_RG = 80
_NRG = _N // _RG
_KRG = -(-_NRG // _NS)


def _edge_body(f1_ref, gate_ref, wa_ref, wb_ref, m_ref):
    q = jnp.dot(f1_ref[...], wa_ref[...], preferred_element_type=jnp.float32)
    f = jnp.dot(q, wb_ref[...], preferred_element_type=jnp.float32)
    g = gate_ref[...]
    m1 = f * g
    m_ref[0] = m1
    m_ref[1] = m1 * g


def _edge_stage(feature1, gate_edge, W_f1a, W_f1b):
    BE = 2000
    return pl.pallas_call(
        _edge_body,
        grid=(_E // BE,),
        in_specs=[
            pl.BlockSpec((BE, _F1), lambda i: (i, 0)),
            pl.BlockSpec((BE, _H), lambda i: (i, 0)),
            pl.BlockSpec((_F1, _MID), lambda i: (0, 0)),
            pl.BlockSpec((_MID, _H), lambda i: (0, 0)),
        ],
        out_specs=pl.BlockSpec((2, BE, _H), lambda i: (0, i, 0)),
        out_shape=jax.ShapeDtypeStruct((2, _E, _H), jnp.float32),
    )(feature1, gate_edge, W_f1a, W_f1b)


def _sc_scatter_body(src_hbm, dst_hbm, m_hbm, x_hbm, out_hbm,
                     srcv, dstv, xrows, mrows, agg, sem):
    c = lax.axis_index("c")
    s = lax.axis_index("s")

    zero = jnp.zeros((16,), jnp.float32)

    def zero_row(r, carry):
        for j in range(8):
            xrows[r, pl.ds(j * 16, 16)] = zero
        return carry

    lax.fori_loop(0, _RG, zero_row, None)

    def zero_grp(k, carry):
        g = k * _NS + s

        @pl.when(g < _NRG)
        def _z():
            pltpu.sync_copy(xrows.at[pl.ds(0, _RG)],
                            agg.at[pl.ds(g * _RG, _RG)])

        return carry

    lax.fori_loop(0, _KRG, zero_grp, None)
    plsc.subcore_barrier()

    def chunk_body(k, carry):
        g = k * _NS + s

        @pl.when(g < _NCHUNK)
        def _p():
            base = g * _CB
            pltpu.sync_copy(src_hbm.at[pl.ds(base, _CB)], srcv)
            pltpu.sync_copy(dst_hbm.at[pl.ds(base, _CB)], dstv)
            pltpu.sync_copy(m_hbm.at[c, pl.ds(base, _CB)], mrows)
            pltpu.async_copy(x_hbm.at[srcv], xrows, sem).wait()

            def mul_row(r, cc):
                for j in range(8):
                    sl = pl.ds(j * 16, 16)
                    mrows[r, sl] = mrows[r, sl] * xrows[r, sl]
                return cc

            lax.fori_loop(0, _CB, mul_row, None)
            pltpu.sync_copy(mrows, agg.at[dstv], add=True)

        return carry

    lax.fori_loop(0, _KMAX, chunk_body, None)
    plsc.subcore_barrier()

    def out_grp(k, carry):
        g = k * _NS + s

        @pl.when(g < _NRG)
        def _o():
            pltpu.sync_copy(agg.at[pl.ds(g * _RG, _RG)],
                            out_hbm.at[c, pl.ds(g * _RG, _RG)])

        return carry

    lax.fori_loop(0, _KRG, out_grp, None)


def _scatter_stage(src1d, dst1d, m, x):
    f = pl.kernel(
        _sc_scatter_body,
        out_type=jax.ShapeDtypeStruct((_NC, _AROWS, _H), jnp.float32),
        mesh=plsc.VectorSubcoreMesh(core_axis_name="c", subcore_axis_name="s",
                                    num_cores=_NC, num_subcores=_NS),
        scratch_types=[
            pltpu.VMEM((_CB,), jnp.int32),
            pltpu.VMEM((_CB,), jnp.int32),
            pltpu.VMEM((_CB, _H), jnp.float32),
            pltpu.VMEM((_CB, _H), jnp.float32),
            pltpu.VMEM_SHARED((_AROWS, _H), jnp.float32),
            pltpu.SemaphoreType.DMA,
        ],
    )
    return f(src1d, dst1d, m, x)


def _leaky(v):
    return jnp.where(v > 0, v, 0.1 * v)


def _node_body(x_ref, a1_ref, a2_ref, batch_ref,
               wr1_ref, br1_ref, wo1_ref, wr2_ref, br2_ref, wo2_ref,
               wl1_ref, bl1_ref, wl2_ref, bl2_ref, wc_ref, bc_ref,
               gw_ref, gb_ref, gms_ref, out_ref):
    x = x_ref[...]
    dot = functools.partial(jnp.dot, preferred_element_type=jnp.float32)
    h1 = dot(a1_ref[0], wr1_ref[...]) + br1_ref[...] + dot(x, wo1_ref[...])
    h1 = _leaky(dot(h1, wl1_ref[...]) + bl1_ref[...])
    h2 = dot(a2_ref[0], wr2_ref[...]) + br2_ref[...] + dot(x, wo2_ref[...])
    h2 = _leaky(dot(h2, wl2_ref[...]) + bl2_ref[...])
    wc = wc_ref[...]
    h = dot(h1, wc[:_H, :]) + dot(h2, wc[_H:, :]) + bc_ref[...]

    batch = batch_ref[...]
    seg = lax.broadcasted_iota(jnp.int32, (_G, _N), 0)
    onehot = (seg == batch).astype(jnp.float32)
    cnt = jnp.maximum(jnp.sum(onehot, axis=1, keepdims=True), 1.0)
    mean = dot(onehot, h) / cnt
    mean_n = lax.dot_general(onehot, mean, (((0,), (0,)), ((), ())),
                             preferred_element_type=jnp.float32)
    out = h - mean_n * gms_ref[...]
    var = dot(onehot, out * out) / cnt
    var_n = lax.dot_general(onehot, var, (((0,), (0,)), ((), ())),
                            preferred_element_type=jnp.float32)
    out_ref[...] = gw_ref[...] * out * lax.rsqrt(var_n + 1e-5) + gb_ref[...]


def _node_stage(x, agg, batch,
                W_rel1, b_rel1, W_root1, W_rel2, b_rel2, W_root2,
                W_lin1, b_lin1, W_lin2, b_lin2, W_cat, b_cat,
                gn_weight, gn_bias, gn_mean_scale):
    row = lambda v: v.reshape(1, -1)

    def full(a):
        nd = a.ndim
        return pl.BlockSpec(a.shape, lambda i: (0,) * nd)

    args = (x, agg, agg, batch.reshape(1, _N),
            W_rel1, row(b_rel1), W_root1, W_rel2, row(b_rel2), W_root2,
            W_lin1, row(b_lin1), W_lin2, row(b_lin2), W_cat, row(b_cat),
            row(gn_weight), row(gn_bias), row(gn_mean_scale))
    specs = [full(a) for a in args]
    specs[1] = pl.BlockSpec((1, _N, _H), lambda i: (0, 0, 0))
    specs[2] = pl.BlockSpec((1, _N, _H), lambda i: (1, 0, 0))
    return pl.pallas_call(
        _node_body,
        grid=(1,),
        in_specs=specs,
        out_specs=pl.BlockSpec((_N, _H), lambda i: (0, 0)),
        out_shape=jax.ShapeDtypeStruct((_N, _H), jnp.float32),
    )(*args)


def kernel(x, gate_edge, feature1, feature2, edge_index, batch,
           W_f1a, W_f1b, W_f2a, W_f2b,
           W_rel1, b_rel1, W_root1, W_rel2, b_rel2, W_root2,
           W_lin1, b_lin1, W_lin2, b_lin2, W_cat, b_cat,
           gn_weight, gn_bias, gn_mean_scale):
    src1d = edge_index[0]
    dst1d = edge_index[1]
    m = _edge_stage(feature1, gate_edge, W_f1a, W_f1b)
    agg = _scatter_stage(src1d, dst1d, m, x)
    return _node_stage(x, agg, batch,
                       W_rel1, b_rel1, W_root1, W_rel2, b_rel2, W_root2,
                       W_lin1, b_lin1, W_lin2, b_lin2, W_cat, b_cat,
                       gn_weight, gn_bias, gn_mean_scale)
